# Initial kernel scaffold; baseline (speedup 1.0000x reference)
import jax
import jax.numpy as jnp
from jax.experimental import pallas as pl


def kernel(x, f_idx, t_idx):
    raise NotImplementedError("write your pallas kernel here")


# speedup vs baseline: 3.3903x; 3.3903x over previous
import jax
import jax.numpy as jnp
from jax import lax
from jax.experimental import pallas as pl
from jax.experimental.pallas import tpu as pltpu
from jax.experimental.pallas import tpu_sc as plsc

B, F, T, K = 4096, 64, 128, 64
NC, NS, L = 2, 16, 16
NW = NC * NS
BPW = B // NW
ROW = F * T


def _sc_body(x_ref, f_ref, t_ref, out_ref, f_v, t_v, off_v, idx_v, val_v,
             out_v, sem):
    wid = lax.axis_index("s") * NC + lax.axis_index("c")
    base = wid * BPW

    pltpu.sync_copy(f_ref, f_v)
    pltpu.sync_copy(t_ref, t_v)

    for k16 in range(K // L):
        sl = pl.ds(k16 * L, L)
        off_v[sl] = f_v[sl] * T + t_v[sl]

    lanes = lax.iota(jnp.int32, L)
    row_base = [(base + b8 * L + lanes) * ROW for b8 in range(BPW // L)]

    def build(k, carry):
        off_k = off_v[pl.ds(k, L)][0]
        for b8 in range(BPW // L):
            idx_v[pl.ds(k * BPW + b8 * L, L)] = row_base[b8] + off_k
        return carry

    lax.fori_loop(0, K, build, 0)

    pltpu.async_copy(x_ref.at[idx_v], val_v, sem).wait()

    for b8 in range(BPW // L):

        def red(k, acc):
            v = val_v[pl.ds(k * BPW + b8 * L, L)]
            return acc + v * v

        out_v[pl.ds(b8 * L, L)] = lax.fori_loop(
            0, K, red, jnp.zeros((L,), jnp.float32))

    pltpu.sync_copy(out_v, out_ref.at[pl.ds(base, BPW)])


@jax.jit
def kernel(x, f_idx, t_idx):
    kern = pl.kernel(
        _sc_body,
        out_type=jax.ShapeDtypeStruct((B,), jnp.float32),
        mesh=plsc.VectorSubcoreMesh(core_axis_name="c", subcore_axis_name="s"),
        scratch_types=[
            pltpu.VMEM((K,), jnp.int32),
            pltpu.VMEM((K,), jnp.int32),
            pltpu.VMEM((K + L,), jnp.int32),
            pltpu.VMEM((K * BPW,), jnp.int32),
            pltpu.VMEM((K * BPW,), jnp.float32),
            pltpu.VMEM((BPW,), jnp.float32),
            pltpu.SemaphoreType.DMA,
        ],
    )
    return kern(x.reshape(-1), f_idx, t_idx)


# speedup vs baseline: 3.6931x; 1.0893x over previous
import jax
import jax.numpy as jnp
from jax import lax
from jax.experimental import pallas as pl
from jax.experimental.pallas import tpu as pltpu
from jax.experimental.pallas import tpu_sc as plsc

B, F, T, K = 4096, 64, 128, 64
NC, NS, L = 2, 16, 16
NW = NC * NS
BPW = B // NW
ROW = F * T
NCHUNK = 4
KC = K // NCHUNK
NB = BPW // L


def _sc_body(x_ref, f_ref, t_ref, out_ref, f_v, t_v, off_v, idx_v, val_v,
             out_v, sems):
    wid = lax.axis_index("s") * NC + lax.axis_index("c")
    base = wid * BPW

    pltpu.sync_copy(f_ref, f_v)
    pltpu.sync_copy(t_ref, t_v)

    for k16 in range(K // L):
        sl = pl.ds(k16 * L, L)
        off_v[sl] = f_v[sl] * T + t_v[sl]

    lanes = lax.iota(jnp.int32, L)
    row_base = [(base + b8 * L + lanes) * ROW for b8 in range(NB)]

    copies = []
    for c in range(NCHUNK):
        def build(k, carry):
            off_k = off_v[pl.ds(k, L)][0]
            for b8 in range(NB):
                idx_v[pl.ds(k * BPW + b8 * L, L)] = row_base[b8] + off_k
            return carry

        lax.fori_loop(c * KC, (c + 1) * KC, build, 0)
        csl = pl.ds(c * KC * BPW, KC * BPW)
        cp = pltpu.async_copy(x_ref.at[idx_v.at[csl]], val_v.at[csl], sems[c])
        copies.append(cp)

    accs = [jnp.zeros((L,), jnp.float32) for _ in range(NB)]
    for c in range(NCHUNK):
        copies[c].wait()

        def red(k, accs):
            out = []
            for b8 in range(NB):
                v = val_v[pl.ds(k * BPW + b8 * L, L)]
                out.append(accs[b8] + v * v)
            return tuple(out)

        accs = lax.fori_loop(c * KC, (c + 1) * KC, red, tuple(accs))

    for b8 in range(NB):
        out_v[pl.ds(b8 * L, L)] = accs[b8]

    pltpu.sync_copy(out_v, out_ref.at[pl.ds(base, BPW)])


@jax.jit
def kernel(x, f_idx, t_idx):
    kern = pl.kernel(
        _sc_body,
        out_type=jax.ShapeDtypeStruct((B,), jnp.float32),
        mesh=plsc.VectorSubcoreMesh(core_axis_name="c", subcore_axis_name="s"),
        scratch_types=[
            pltpu.VMEM((K,), jnp.int32),
            pltpu.VMEM((K,), jnp.int32),
            pltpu.VMEM((K + L,), jnp.int32),
            pltpu.VMEM((K * BPW,), jnp.int32),
            pltpu.VMEM((K * BPW,), jnp.float32),
            pltpu.VMEM((BPW,), jnp.float32),
            [pltpu.SemaphoreType.DMA] * NCHUNK,
        ],
    )
    return kern(x.reshape(-1), f_idx, t_idx)


# speedup vs baseline: 3.7665x; 1.0199x over previous
import jax
import jax.numpy as jnp
from jax import lax
from jax.experimental import pallas as pl
from jax.experimental.pallas import tpu as pltpu
from jax.experimental.pallas import tpu_sc as plsc

B, F, T, K = 4096, 64, 128, 64
NC, NS, L = 2, 16, 16
NW = NC * NS
BPW = B // NW
ROW = F * T
NCHUNK = 4
KC = K // NCHUNK
NB = BPW // L


def _sc_body(x_ref, off_ref, out_ref, off_v, idx_v, val_v, out_v, osem, sems):
    wid = lax.axis_index("s") * NC + lax.axis_index("c")
    base = wid * BPW

    ocp = pltpu.async_copy(off_ref, off_v.at[pl.ds(0, K)], osem)
    lanes = lax.iota(jnp.int32, L)
    row_base = [(base + b8 * L + lanes) * ROW for b8 in range(NB)]
    ocp.wait()

    copies = []
    for c in range(NCHUNK):
        def build(k, carry):
            off_k = off_v[pl.ds(k, L)][0]
            for b8 in range(NB):
                idx_v[pl.ds(k * BPW + b8 * L, L)] = row_base[b8] + off_k
            return carry

        lax.fori_loop(c * KC, (c + 1) * KC, build, 0)
        csl = pl.ds(c * KC * BPW, KC * BPW)
        cp = pltpu.async_copy(x_ref.at[idx_v.at[csl]], val_v.at[csl], sems[c])
        copies.append(cp)

    accs = [jnp.zeros((L,), jnp.float32) for _ in range(NB)]
    for c in range(NCHUNK):
        copies[c].wait()

        def red(k, accs):
            out = []
            for b8 in range(NB):
                v = val_v[pl.ds(k * BPW + b8 * L, L)]
                out.append(accs[b8] + v * v)
            return tuple(out)

        accs = lax.fori_loop(c * KC, (c + 1) * KC, red, tuple(accs))

    for b8 in range(NB):
        out_v[pl.ds(b8 * L, L)] = accs[b8]

    pltpu.sync_copy(out_v, out_ref.at[pl.ds(base, BPW)])


@jax.jit
def kernel(x, f_idx, t_idx):
    off = f_idx * T + t_idx
    kern = pl.kernel(
        _sc_body,
        out_type=jax.ShapeDtypeStruct((B,), jnp.float32),
        mesh=plsc.VectorSubcoreMesh(core_axis_name="c", subcore_axis_name="s"),
        scratch_types=[
            pltpu.VMEM((K + L,), jnp.int32),
            pltpu.VMEM((K * BPW,), jnp.int32),
            pltpu.VMEM((K * BPW,), jnp.float32),
            pltpu.VMEM((BPW,), jnp.float32),
            pltpu.SemaphoreType.DMA,
            [pltpu.SemaphoreType.DMA] * NCHUNK,
        ],
    )
    return kern(x.reshape(-1), off)
